# Initial kernel scaffold; baseline (speedup 1.0000x reference)
import jax
import jax.numpy as jnp
from jax.experimental import pallas as pl


def kernel(snip_feature, seg_lens, params):
    raise NotImplementedError("write your pallas kernel here")


# speedup vs baseline: 13.5276x; 13.5276x over previous
import functools

import jax
import jax.numpy as jnp
from jax import lax
from jax.experimental import pallas as pl
from jax.experimental.pallas import tpu as pltpu
from jax.experimental.pallas import tpu_sc as plsc

B, C, T = 4, 256, 1024
K = 3
F32 = jnp.float32


def _mm(x, w):
    return lax.dot_general(x, w, (((1,), (1,)), ((), ())),
                           preferred_element_type=F32)


def _block_diag(w, groups):
    o, ig = w.shape
    og = o // groups
    r = w.reshape(groups, og, ig)
    eye = jnp.eye(groups, dtype=w.dtype)
    return jnp.einsum('goi,gh->gohi', r, eye).reshape(o, groups * ig)


def _k0_body(xp_ref, w_ref, b_ref, out_ref):
    xp = xp_ref[0]
    acc = _mm(xp[0:T], w_ref[0])
    acc += _mm(xp[1:T + 1], w_ref[1])
    acc += _mm(xp[2:T + 2], w_ref[2])
    out_ref[0] = jax.nn.relu(acc + b_ref[...])


def _backbone(xp, wd, bias):
    return pl.pallas_call(
        _k0_body,
        grid=(B,),
        in_specs=[
            pl.BlockSpec((1, T + 2, C), lambda b: (b, 0, 0)),
            pl.BlockSpec((3, C, C), lambda b: (0, 0, 0)),
            pl.BlockSpec((1, C), lambda b: (0, 0)),
        ],
        out_specs=pl.BlockSpec((1, T, C), lambda b: (b, 0, 0)),
        out_shape=jax.ShapeDtypeStruct((B, T, C), F32),
    )(xp, wd, bias)


def _k1_body(x_ref, wt1_ref, wt2_ref, wt3_ref, wa_ref, wb_ref,
             bt1_ref, bt2_ref, bt3_ref, bs1_ref,
             tpi_ref, ya_ref, ybp_ref):
    x = x_ref[0]
    t1 = jax.nn.relu(_mm(x, wt1_ref[...]) + bt1_ref[...])
    z = jnp.zeros((1, 128), F32)
    t1p = jnp.concatenate([z, t1, z], axis=0)
    t2 = _mm(t1p[0:T], wt2_ref[0])
    t2 += _mm(t1p[1:T + 1], wt2_ref[1])
    t2 += _mm(t1p[2:T + 2], wt2_ref[2])
    t2 = jax.nn.relu(t2 + bt2_ref[...])
    t3 = _mm(t2, wt3_ref[...]) + bt3_ref[...]
    tpi_ref[0] = x + t3
    ya_ref[0] = _mm(x, wa_ref[...])
    ybp_ref[0] = _mm(x, wb_ref[...]) + bs1_ref[...]


def _k1(x, wt1, wt2d, wt3, wa, wb, bt1, bt2, bt3, bs1):
    full = lambda shape: pl.BlockSpec(shape, lambda b: (0,) * len(shape))
    return pl.pallas_call(
        _k1_body,
        grid=(B,),
        in_specs=[
            pl.BlockSpec((1, T, C), lambda b: (b, 0, 0)),
            full((128, 256)), full((3, 128, 128)), full((256, 128)),
            full((128, 256)), full((128, 256)),
            full((1, 128)), full((1, 128)), full((1, 256)), full((1, 128)),
        ],
        out_specs=[
            pl.BlockSpec((1, T, C), lambda b: (b, 0, 0)),
            pl.BlockSpec((1, T, 128), lambda b: (b, 0, 0)),
            pl.BlockSpec((1, T, 128), lambda b: (b, 0, 0)),
        ],
        out_shape=[
            jax.ShapeDtypeStruct((B, T, C), F32),
            jax.ShapeDtypeStruct((B, T, 128), F32),
            jax.ShapeDtypeStruct((B, T, 128), F32),
        ],
    )(x, wt1, wt2d, wt3, wa, wb, bt1, bt2, bt3, bs1)


def _k2_body(x_ref, valid_ref, i0_ref, i1_ref, i2_ref):
    b = pl.program_id(0)
    x = x_ref[0]
    g = lax.dot_general(x, x, (((1,), (1,)), ((), ())),
                        preferred_element_type=F32)
    xx = jnp.sum(x * x, axis=1)
    pd = 2.0 * g - xx[:, None] - xx[None, :]
    pd = jnp.where(valid_ref[0] > 0.5, pd, -1e9)
    iota_c = lax.broadcasted_iota(jnp.int32, (T, T), 1)
    offs = b * T
    outs = (i0_ref, i1_ref, i2_ref)
    for j in range(K):
        m = jnp.max(pd, axis=1)
        idxj = jnp.min(jnp.where(pd == m[:, None], iota_c, T), axis=1)
        outs[j][0] = (idxj + offs).reshape(1, T)
        pd = jnp.where(iota_c == idxj[:, None], -jnp.inf, pd)


def _k2(x, valid):
    ispec = pl.BlockSpec((1, 1, T), lambda b: (b, 0, 0))
    return pl.pallas_call(
        _k2_body,
        grid=(B,),
        in_specs=[
            pl.BlockSpec((1, T, C), lambda b: (b, 0, 0)),
            pl.BlockSpec((1, 1, T), lambda b: (b, 0, 0)),
        ],
        out_specs=[ispec, ispec, ispec],
        out_shape=[jax.ShapeDtypeStruct((B, 1, T), jnp.int32)] * 3,
    )(x, valid)


_NG = B * T * K
_D = 128


def _sc_gather(table, idx):
    info = plsc.get_sparse_core_info()
    nc, ns = info.num_cores, info.num_subcores
    nw = nc * ns
    b_per_w = _NG // nw
    mesh = plsc.VectorSubcoreMesh(core_axis_name="c", subcore_axis_name="s")

    @functools.partial(
        pl.kernel, mesh=mesh,
        out_type=jax.ShapeDtypeStruct((_NG, _D), F32),
        scratch_types=[
            pltpu.VMEM((b_per_w,), jnp.int32),
            pltpu.VMEM((b_per_w, _D), F32),
            pltpu.SemaphoreType.DMA,
        ],
    )
    def gk(table_hbm, idx_hbm, out_hbm, idx_v, rows_v, sem):
        wid = lax.axis_index("s") * nc + lax.axis_index("c")
        base = wid * b_per_w
        pltpu.sync_copy(idx_hbm.at[pl.ds(base, b_per_w)], idx_v)
        pltpu.async_copy(table_hbm.at[idx_v], rows_v, sem).wait()
        pltpu.sync_copy(rows_v, out_hbm.at[pl.ds(base, b_per_w)])

    return gk(table, idx)


def _k3_body(yg_ref, ybp_ref, tpi_ref, w2_ref, w3_ref, b2_ref, out_ref):
    ybp = ybp_ref[0]
    smax = None
    for j in range(K):
        s1 = jax.nn.relu(yg_ref[j, 0] + ybp)
        s2 = jax.nn.relu(_mm(s1, w2_ref[...]) + b2_ref[...])
        s3 = _mm(s2, w3_ref[...])
        smax = s3 if smax is None else jnp.maximum(smax, s3)
    out_ref[0] = jax.nn.relu(tpi_ref[0] + smax)


def _k3(yg, ybp, tpi, w2, w3, b2, b3):
    full = lambda shape: pl.BlockSpec(shape, lambda b: (0,) * len(shape))
    return pl.pallas_call(
        _k3_body,
        grid=(B,),
        in_specs=[
            pl.BlockSpec((K, 1, T, 128), lambda b: (0, b, 0, 0)),
            pl.BlockSpec((1, T, 128), lambda b: (b, 0, 0)),
            pl.BlockSpec((1, T, C), lambda b: (b, 0, 0)),
            full((128, 128)), full((256, 128)), full((1, 128)),
        ],
        out_specs=pl.BlockSpec((1, T, C), lambda b: (b, 0, 0)),
        out_shape=jax.ShapeDtypeStruct((B, T, C), F32),
    )(yg, ybp, tpi, w2, w3, b2)


def _gcnext(x, valid, p):
    wt1 = p['wt1'][:, :, 0]
    wt2d = jnp.stack(
        [_block_diag(p['wt2'][:, :, d], 32) for d in range(3)], axis=0)
    wt3 = p['wt3'][:, :, 0]
    wa = p['ws1'][:, :C, 0, 0]
    wb = p['ws1'][:, C:, 0, 0]
    w2 = _block_diag(p['ws2'][:, :, 0, 0], 32)
    w3 = p['ws3'][:, :, 0, 0]
    bt1 = p['bt1'].reshape(1, 128)
    bt2 = p['bt2'].reshape(1, 128)
    bt3 = (p['bt3'] + p['bs3']).reshape(1, 256)
    bs1 = p['bs1'].reshape(1, 128)
    b2 = p['bs2'].reshape(1, 128)

    tpi, ya, ybp = _k1(x, wt1, wt2d, wt3, wa, wb, bt1, bt2, bt3, bs1)
    i0, i1, i2 = _k2(x, valid)
    idx = jnp.stack([i0[:, 0, :], i1[:, 0, :], i2[:, 0, :]], axis=0)
    gathered = _sc_gather(ya.reshape(B * T, 128), idx.reshape(-1))
    yg = gathered.reshape(K, B, T, 128)
    return _k3(yg, ybp, tpi, w2, w3, b2, None)


def kernel(snip_feature, seg_lens, params):
    xt = jnp.swapaxes(snip_feature, 1, 2)
    xp = jnp.pad(xt, ((0, 0), (1, 1), (0, 0)))
    wbd = jnp.stack(
        [_block_diag(params['w_b'][:, :, d], 4) for d in range(3)], axis=0)
    x = _backbone(xp, wbd, params['b_b'].reshape(1, C))

    valid = (jnp.arange(T)[None, :] < seg_lens[:, None]).astype(F32)
    valid = valid.reshape(B, 1, T)

    x = _gcnext(x, valid, params['g1'])
    x = _gcnext(x, valid, params['g2'])
    return jnp.swapaxes(x, 1, 2)
